# baseline (speedup 1.0000x reference)
import jax
import jax.numpy as jnp
from jax.experimental import pallas as pl
from jax.experimental.pallas import tpu as pltpu

H, KVH, DH = 16, 8, 128
EPS = 1e-6
NEG = -1e30
LOG2E = 1.4426950408889634

BM_PROJ = 256
BM_OUT = 512
BM_Q = 512
GQ = H // KVH


def _rms_norm_heads(t, w, extra_scale):
    rows = t.shape[0]
    n = t.shape[1] // DH
    t3 = t.reshape(rows, n, DH)
    var = jnp.mean(t3 * t3, axis=-1, keepdims=True)
    t3 = t3 * (jax.lax.rsqrt(var + EPS) * extra_scale)
    return (t3 * w.reshape(1, 1, DH)).reshape(rows, n * DH)


def _qkv_proj_kernel(x_ref, wq_ref, wk_ref, wv_ref, qw_ref, kw_ref,
                     q_ref, k_ref, v_ref, wqb_ref, wkb_ref, wvb_ref):
    @pl.when(pl.program_id(0) == 0)
    def _cast_weights():
        wqb_ref[...] = wq_ref[...].astype(jnp.bfloat16)
        wkb_ref[...] = wk_ref[...].astype(jnp.bfloat16)
        wvb_ref[...] = wv_ref[...].astype(jnp.bfloat16)

    xb = x_ref[...].astype(jnp.bfloat16)
    dims = (((1,), (1,)), ((), ()))
    q = jax.lax.dot_general(xb, wqb_ref[...], dims,
                            preferred_element_type=jnp.float32)
    k = jax.lax.dot_general(xb, wkb_ref[...], dims,
                            preferred_element_type=jnp.float32)
    v = jax.lax.dot_general(xb, wvb_ref[...], dims,
                            preferred_element_type=jnp.float32)
    qn = _rms_norm_heads(q, qw_ref[...], DH ** -0.5 * LOG2E)
    kn = _rms_norm_heads(k, kw_ref[...], 1.0)
    q_ref[...] = qn.astype(jnp.bfloat16)
    k_ref[...] = kn.astype(jnp.bfloat16)
    v_ref[...] = v.astype(jnp.bfloat16)


def _attn_out_kernel(q_ref, k_ref, v_ref, wo_ref, o_ref, attn_ref, wob_ref):
    i = pl.program_id(0)
    g = pl.program_id(1)
    dims_nt = (((1,), (1,)), ((), ()))
    dims_nn = (((1,), (0,)), ((), ()))

    @pl.when((i == 0) & (g == 0))
    def _cast_weights():
        wob_ref[...] = wo_ref[...].astype(jnp.bfloat16)
    qcat = jnp.concatenate([q_ref[:, :DH], q_ref[:, DH:]], axis=0)

    def chunk(j, carry, mask):
        acc, l = carry
        kc = k_ref[pl.ds(j * BM_Q, BM_Q), :]
        s = jax.lax.dot_general(qcat, kc, dims_nt,
                                preferred_element_type=jnp.float32)
        if mask:
            row = jax.lax.broadcasted_iota(
                jnp.int32, (2 * BM_Q, BM_Q), 0) % BM_Q
            col = jax.lax.broadcasted_iota(
                jnp.int32, (2 * BM_Q, BM_Q), 1)
            s = jnp.where(row >= col, s, NEG)
        p = jnp.exp2(s)
        l = l + jnp.sum(p, axis=-1, keepdims=True)
        vc = v_ref[pl.ds(j * BM_Q, BM_Q), :]
        acc = acc + jax.lax.dot_general(p.astype(jnp.bfloat16), vc, dims_nn,
                                        preferred_element_type=jnp.float32)
        return acc, l

    zero = (jnp.zeros((2 * BM_Q, DH), jnp.float32),
            jnp.zeros((2 * BM_Q, 1), jnp.float32))
    acc, l = jax.lax.fori_loop(
        0, i, lambda j, c: chunk(j, c, False), zero)
    acc, l = chunk(i, (acc, l), True)
    o = (acc / l).astype(jnp.bfloat16)
    oh = jnp.concatenate([o[:BM_Q], o[BM_Q:]], axis=1)
    for gg in range(KVH):
        @pl.when(g == gg)
        def _store(gg=gg):
            attn_ref[:, gg * GQ * DH:(gg + 1) * GQ * DH] = oh

    @pl.when(g == KVH - 1)
    def _out_proj():
        o_ref[...] = jax.lax.dot_general(
            attn_ref[...], wob_ref[...], (((1,), (1,)), ((), ())),
            preferred_element_type=jnp.float32)


def kernel(x, Wq, Wk, Wv, Wo, q_norm_w, k_norm_w):
    b, s, d = x.shape
    x2 = x.reshape(s, d)
    qw = q_norm_w.reshape(1, DH)
    kw = k_norm_w.reshape(1, DH)

    n_row_blocks = s // BM_PROJ
    q, k, v = pl.pallas_call(
        _qkv_proj_kernel,
        grid=(n_row_blocks,),
        in_specs=[
            pl.BlockSpec((BM_PROJ, d), lambda i: (i, 0)),
            pl.BlockSpec((H * DH, d), lambda i: (0, 0)),
            pl.BlockSpec((KVH * DH, d), lambda i: (0, 0)),
            pl.BlockSpec((KVH * DH, d), lambda i: (0, 0)),
            pl.BlockSpec((1, DH), lambda i: (0, 0)),
            pl.BlockSpec((1, DH), lambda i: (0, 0)),
        ],
        out_specs=[
            pl.BlockSpec((BM_PROJ, H * DH), lambda i: (i, 0)),
            pl.BlockSpec((BM_PROJ, KVH * DH), lambda i: (i, 0)),
            pl.BlockSpec((BM_PROJ, KVH * DH), lambda i: (i, 0)),
        ],
        out_shape=[
            jax.ShapeDtypeStruct((s, H * DH), jnp.bfloat16),
            jax.ShapeDtypeStruct((s, KVH * DH), jnp.bfloat16),
            jax.ShapeDtypeStruct((s, KVH * DH), jnp.bfloat16),
        ],
        scratch_shapes=[
            pltpu.VMEM((H * DH, d), jnp.bfloat16),
            pltpu.VMEM((KVH * DH, d), jnp.bfloat16),
            pltpu.VMEM((KVH * DH, d), jnp.bfloat16),
        ],
    )(x2, Wq, Wk, Wv, qw, kw)

    n_q_blocks = s // BM_Q
    out = pl.pallas_call(
        _attn_out_kernel,
        grid=(n_q_blocks, KVH),
        in_specs=[
            pl.BlockSpec((BM_Q, GQ * DH), lambda i, g: (i, g)),
            pl.BlockSpec((s, DH), lambda i, g: (0, g)),
            pl.BlockSpec((s, DH), lambda i, g: (0, g)),
            pl.BlockSpec((d, H * DH), lambda i, g: (0, 0)),
        ],
        out_specs=pl.BlockSpec((BM_Q, d), lambda i, g: (i, 0)),
        out_shape=jax.ShapeDtypeStruct((s, d), jnp.float32),
        scratch_shapes=[
            pltpu.VMEM((BM_Q, H * DH), jnp.bfloat16),
            pltpu.VMEM((d, H * DH), jnp.bfloat16),
        ],
    )(q, k, v, Wo)

    return out.reshape(b, s, d)


# speedup vs baseline: 1.0079x; 1.0079x over previous
import jax
import jax.numpy as jnp
from jax.experimental import pallas as pl
from jax.experimental.pallas import tpu as pltpu

H, KVH, DH = 16, 8, 128
EPS = 1e-6
NEG = -1e30
LOG2E = 1.4426950408889634

BM_PROJ = 256
BM_OUT = 512
BM_Q = 512
GQ = H // KVH


def _rms_norm_heads(t, w, extra_scale):
    rows = t.shape[0]
    n = t.shape[1] // DH
    t3 = t.reshape(rows, n, DH)
    var = jnp.mean(t3 * t3, axis=-1, keepdims=True)
    t3 = t3 * (jax.lax.rsqrt(var + EPS) * extra_scale)
    return (t3 * w.reshape(1, 1, DH)).reshape(rows, n * DH)


def _qkv_proj_kernel(x_ref, wq_ref, wk_ref, wv_ref, qw_ref, kw_ref,
                     q_ref, k_ref, v_ref, wqb_ref, wkb_ref, wvb_ref):
    @pl.when(pl.program_id(0) == 0)
    def _cast_weights():
        wqb_ref[...] = wq_ref[...].astype(jnp.bfloat16)
        wkb_ref[...] = wk_ref[...].astype(jnp.bfloat16)
        wvb_ref[...] = wv_ref[...].astype(jnp.bfloat16)

    xb = x_ref[...].astype(jnp.bfloat16)
    dims = (((1,), (1,)), ((), ()))
    q = jax.lax.dot_general(xb, wqb_ref[...], dims,
                            preferred_element_type=jnp.float32)
    k = jax.lax.dot_general(xb, wkb_ref[...], dims,
                            preferred_element_type=jnp.float32)
    v = jax.lax.dot_general(xb, wvb_ref[...], dims,
                            preferred_element_type=jnp.float32)
    qn = _rms_norm_heads(q, qw_ref[...], DH ** -0.5 * LOG2E)
    kn = _rms_norm_heads(k, kw_ref[...], 1.0)
    q_ref[...] = qn.astype(jnp.bfloat16)
    k_ref[...] = kn.astype(jnp.bfloat16)
    v_ref[...] = v.astype(jnp.bfloat16)


def _attn_kernel(q_ref, k_ref, v_ref, o_ref):
    i = pl.program_id(1)
    dims_nt = (((1,), (1,)), ((), ()))
    dims_nn = (((1,), (0,)), ((), ()))
    qcat = jnp.concatenate([q_ref[:, :DH], q_ref[:, DH:]], axis=0)

    def chunk(j, carry, mask):
        acc, l = carry
        kc = k_ref[pl.ds(j * BM_Q, BM_Q), :]
        s = jax.lax.dot_general(qcat, kc, dims_nt,
                                preferred_element_type=jnp.float32)
        if mask:
            row = jax.lax.broadcasted_iota(
                jnp.int32, (2 * BM_Q, BM_Q), 0) % BM_Q
            col = jax.lax.broadcasted_iota(
                jnp.int32, (2 * BM_Q, BM_Q), 1)
            s = jnp.where(row >= col, s, NEG)
        p = jnp.exp2(s)
        l = l + jnp.sum(p, axis=-1, keepdims=True)
        vc = v_ref[pl.ds(j * BM_Q, BM_Q), :]
        acc = acc + jax.lax.dot_general(p.astype(jnp.bfloat16), vc, dims_nn,
                                        preferred_element_type=jnp.float32)
        return acc, l

    zero = (jnp.zeros((2 * BM_Q, DH), jnp.float32),
            jnp.zeros((2 * BM_Q, 1), jnp.float32))
    acc, l = jax.lax.fori_loop(
        0, i, lambda j, c: chunk(j, c, False), zero)
    acc, l = chunk(i, (acc, l), True)
    o = (acc / l).astype(jnp.bfloat16)
    o_ref[:, :DH] = o[:BM_Q]
    o_ref[:, DH:] = o[BM_Q:]


def _out_proj_kernel(a_ref, wo_ref, o_ref, wob_ref):
    @pl.when(pl.program_id(0) == 0)
    def _cast_weights():
        wob_ref[...] = wo_ref[...].astype(jnp.bfloat16)

    o_ref[...] = jax.lax.dot_general(
        a_ref[...], wob_ref[...], (((1,), (1,)), ((), ())),
        preferred_element_type=jnp.float32)


def kernel(x, Wq, Wk, Wv, Wo, q_norm_w, k_norm_w):
    b, s, d = x.shape
    x2 = x.reshape(s, d)
    qw = q_norm_w.reshape(1, DH)
    kw = k_norm_w.reshape(1, DH)

    n_row_blocks = s // BM_PROJ
    q, k, v = pl.pallas_call(
        _qkv_proj_kernel,
        grid=(n_row_blocks,),
        in_specs=[
            pl.BlockSpec((BM_PROJ, d), lambda i: (i, 0)),
            pl.BlockSpec((H * DH, d), lambda i: (0, 0)),
            pl.BlockSpec((KVH * DH, d), lambda i: (0, 0)),
            pl.BlockSpec((KVH * DH, d), lambda i: (0, 0)),
            pl.BlockSpec((1, DH), lambda i: (0, 0)),
            pl.BlockSpec((1, DH), lambda i: (0, 0)),
        ],
        out_specs=[
            pl.BlockSpec((BM_PROJ, H * DH), lambda i: (i, 0)),
            pl.BlockSpec((BM_PROJ, KVH * DH), lambda i: (i, 0)),
            pl.BlockSpec((BM_PROJ, KVH * DH), lambda i: (i, 0)),
        ],
        out_shape=[
            jax.ShapeDtypeStruct((s, H * DH), jnp.bfloat16),
            jax.ShapeDtypeStruct((s, KVH * DH), jnp.bfloat16),
            jax.ShapeDtypeStruct((s, KVH * DH), jnp.bfloat16),
        ],
        scratch_shapes=[
            pltpu.VMEM((H * DH, d), jnp.bfloat16),
            pltpu.VMEM((KVH * DH, d), jnp.bfloat16),
            pltpu.VMEM((KVH * DH, d), jnp.bfloat16),
        ],
    )(x2, Wq, Wk, Wv, qw, kw)

    n_q_blocks = s // BM_Q
    attn = pl.pallas_call(
        _attn_kernel,
        grid=(KVH, n_q_blocks),
        in_specs=[
            pl.BlockSpec((BM_Q, GQ * DH), lambda g, i: (i, g)),
            pl.BlockSpec((s, DH), lambda g, i: (0, g)),
            pl.BlockSpec((s, DH), lambda g, i: (0, g)),
        ],
        out_specs=pl.BlockSpec((BM_Q, GQ * DH), lambda g, i: (i, g)),
        out_shape=jax.ShapeDtypeStruct((s, H * DH), jnp.bfloat16),
    )(q, k, v)

    out = pl.pallas_call(
        _out_proj_kernel,
        grid=(s // BM_OUT,),
        in_specs=[
            pl.BlockSpec((BM_OUT, H * DH), lambda i: (i, 0)),
            pl.BlockSpec((d, H * DH), lambda i: (0, 0)),
        ],
        out_specs=pl.BlockSpec((BM_OUT, d), lambda i: (i, 0)),
        out_shape=jax.ShapeDtypeStruct((s, d), jnp.float32),
        scratch_shapes=[
            pltpu.VMEM((d, H * DH), jnp.bfloat16),
        ],
    )(attn, Wo)

    return out.reshape(b, s, d)


# speedup vs baseline: 1.0669x; 1.0585x over previous
import jax
import jax.numpy as jnp
from jax.experimental import pallas as pl
from jax.experimental.pallas import tpu as pltpu

H, KVH, DH = 16, 8, 128
EPS = 1e-6
NEG = -1e30
LOG2E = 1.4426950408889634

BM_PROJ = 256
BM_OUT = 512
BM_Q = 1024
GQ = H // KVH


def _rms_norm_heads(t, w, extra_scale):
    rows = t.shape[0]
    n = t.shape[1] // DH
    t3 = t.reshape(rows, n, DH)
    var = jnp.mean(t3 * t3, axis=-1, keepdims=True)
    t3 = t3 * (jax.lax.rsqrt(var + EPS) * extra_scale)
    return (t3 * w.reshape(1, 1, DH)).reshape(rows, n * DH)


def _qkv_proj_kernel(x_ref, wq_ref, wk_ref, wv_ref, qw_ref, kw_ref,
                     q_ref, k_ref, v_ref, wqb_ref, wkb_ref, wvb_ref):
    @pl.when(pl.program_id(0) == 0)
    def _cast_weights():
        wqb_ref[...] = wq_ref[...].astype(jnp.bfloat16)
        wkb_ref[...] = wk_ref[...].astype(jnp.bfloat16)
        wvb_ref[...] = wv_ref[...].astype(jnp.bfloat16)

    xb = x_ref[...].astype(jnp.bfloat16)
    dims = (((1,), (1,)), ((), ()))
    q = jax.lax.dot_general(xb, wqb_ref[...], dims,
                            preferred_element_type=jnp.float32)
    k = jax.lax.dot_general(xb, wkb_ref[...], dims,
                            preferred_element_type=jnp.float32)
    v = jax.lax.dot_general(xb, wvb_ref[...], dims,
                            preferred_element_type=jnp.float32)
    qn = _rms_norm_heads(q, qw_ref[...], DH ** -0.5 * LOG2E)
    kn = _rms_norm_heads(k, kw_ref[...], 1.0)
    q_ref[...] = qn.astype(jnp.bfloat16)
    k_ref[...] = kn.astype(jnp.bfloat16)
    v_ref[...] = v.astype(jnp.bfloat16)


def _attn_kernel(q_ref, k_ref, v_ref, o_ref):
    i = pl.program_id(1)
    dims_nt = (((1,), (1,)), ((), ()))
    dims_nn = (((1,), (0,)), ((), ()))
    qcat = jnp.concatenate([q_ref[:, :DH], q_ref[:, DH:]], axis=0)

    def chunk(j, carry, mask):
        acc, l = carry
        kc = k_ref[pl.ds(j * BM_Q, BM_Q), :]
        s = jax.lax.dot_general(qcat, kc, dims_nt,
                                preferred_element_type=jnp.float32)
        if mask:
            row = jax.lax.broadcasted_iota(
                jnp.int32, (2 * BM_Q, BM_Q), 0) % BM_Q
            col = jax.lax.broadcasted_iota(
                jnp.int32, (2 * BM_Q, BM_Q), 1)
            s = jnp.where(row >= col, s, NEG)
        p = jnp.exp2(s)
        l = l + jnp.sum(p, axis=-1, keepdims=True)
        vc = v_ref[pl.ds(j * BM_Q, BM_Q), :]
        acc = acc + jax.lax.dot_general(p.astype(jnp.bfloat16), vc, dims_nn,
                                        preferred_element_type=jnp.float32)
        return acc, l

    zero = (jnp.zeros((2 * BM_Q, DH), jnp.float32),
            jnp.zeros((2 * BM_Q, 1), jnp.float32))
    acc, l = jax.lax.fori_loop(
        0, i, lambda j, c: chunk(j, c, False), zero)
    acc, l = chunk(i, (acc, l), True)
    o = (acc / l).astype(jnp.bfloat16)
    o_ref[:, :DH] = o[:BM_Q]
    o_ref[:, DH:] = o[BM_Q:]


def _out_proj_kernel(a_ref, wo_ref, o_ref, wob_ref):
    @pl.when(pl.program_id(0) == 0)
    def _cast_weights():
        wob_ref[...] = wo_ref[...].astype(jnp.bfloat16)

    o_ref[...] = jax.lax.dot_general(
        a_ref[...], wob_ref[...], (((1,), (1,)), ((), ())),
        preferred_element_type=jnp.float32)


def kernel(x, Wq, Wk, Wv, Wo, q_norm_w, k_norm_w):
    b, s, d = x.shape
    x2 = x.reshape(s, d)
    qw = q_norm_w.reshape(1, DH)
    kw = k_norm_w.reshape(1, DH)

    n_row_blocks = s // BM_PROJ
    q, k, v = pl.pallas_call(
        _qkv_proj_kernel,
        grid=(n_row_blocks,),
        in_specs=[
            pl.BlockSpec((BM_PROJ, d), lambda i: (i, 0)),
            pl.BlockSpec((H * DH, d), lambda i: (0, 0)),
            pl.BlockSpec((KVH * DH, d), lambda i: (0, 0)),
            pl.BlockSpec((KVH * DH, d), lambda i: (0, 0)),
            pl.BlockSpec((1, DH), lambda i: (0, 0)),
            pl.BlockSpec((1, DH), lambda i: (0, 0)),
        ],
        out_specs=[
            pl.BlockSpec((BM_PROJ, H * DH), lambda i: (i, 0)),
            pl.BlockSpec((BM_PROJ, KVH * DH), lambda i: (i, 0)),
            pl.BlockSpec((BM_PROJ, KVH * DH), lambda i: (i, 0)),
        ],
        out_shape=[
            jax.ShapeDtypeStruct((s, H * DH), jnp.bfloat16),
            jax.ShapeDtypeStruct((s, KVH * DH), jnp.bfloat16),
            jax.ShapeDtypeStruct((s, KVH * DH), jnp.bfloat16),
        ],
        scratch_shapes=[
            pltpu.VMEM((H * DH, d), jnp.bfloat16),
            pltpu.VMEM((KVH * DH, d), jnp.bfloat16),
            pltpu.VMEM((KVH * DH, d), jnp.bfloat16),
        ],
    )(x2, Wq, Wk, Wv, qw, kw)

    n_q_blocks = s // BM_Q
    attn = pl.pallas_call(
        _attn_kernel,
        grid=(KVH, n_q_blocks),
        in_specs=[
            pl.BlockSpec((BM_Q, GQ * DH), lambda g, i: (i, g)),
            pl.BlockSpec((s, DH), lambda g, i: (0, g)),
            pl.BlockSpec((s, DH), lambda g, i: (0, g)),
        ],
        out_specs=pl.BlockSpec((BM_Q, GQ * DH), lambda g, i: (i, g)),
        out_shape=jax.ShapeDtypeStruct((s, H * DH), jnp.bfloat16),
    )(q, k, v)

    out = pl.pallas_call(
        _out_proj_kernel,
        grid=(s // BM_OUT,),
        in_specs=[
            pl.BlockSpec((BM_OUT, H * DH), lambda i: (i, 0)),
            pl.BlockSpec((d, H * DH), lambda i: (0, 0)),
        ],
        out_specs=pl.BlockSpec((BM_OUT, d), lambda i: (i, 0)),
        out_shape=jax.ShapeDtypeStruct((s, d), jnp.float32),
        scratch_shapes=[
            pltpu.VMEM((d, H * DH), jnp.bfloat16),
        ],
    )(attn, Wo)

    return out.reshape(b, s, d)
